# baseline (speedup 1.0000x reference)
import functools

import jax
import jax.numpy as jnp
from jax import lax
from jax.experimental import pallas as pl
from jax.experimental.pallas import tpu as pltpu
from jax.experimental.pallas import tpu_sc as plsc

NC = 2
NS = 16
NW = NC * NS

ALPHA = 0.2
KPROP = 20


@functools.lru_cache(maxsize=None)
def _make_propagate(n, d, e, dtype=jnp.float32):
    epw = e // NW
    B = 128
    nfull = epw // B
    tail = epw - nfull * B
    pairs = nfull // 2
    odd = nfull - pairs * 2
    rows_a = (n // NS) // 8 * 8
    rem = n - rows_a * NS
    mesh = plsc.VectorSubcoreMesh(core_axis_name="c", subcore_axis_name="s")

    def tile_rows_copy(src, dst, s):
        pltpu.sync_copy(src.at[pl.ds(s * rows_a, rows_a)],
                        dst.at[pl.ds(s * rows_a, rows_a)])
        if rem:
            @pl.when(s == NS - 1)
            def _():
                pltpu.sync_copy(src.at[pl.ds(rows_a * NS, rem)],
                                dst.at[pl.ds(rows_a * NS, rem)])

    @functools.partial(
        pl.kernel,
        out_type=[jax.ShapeDtypeStruct((n, d), dtype) for _ in range(NC)],
        mesh=mesh,
        scratch_types=[
            pltpu.VMEM((epw,), jnp.int32),
            pltpu.VMEM((B,), jnp.int32),
            pltpu.VMEM((B,), jnp.int32),
            pltpu.VMEM((tail if tail else 8,), jnp.int32),
            pltpu.VMEM((2, B, d), dtype),
            pltpu.VMEM((tail if tail else 8, d), dtype),
            pltpu.VMEM_SHARED((n, d), dtype),
            pltpu.SemaphoreType.DMA,
            pltpu.SemaphoreType.DMA,
            pltpu.SemaphoreType.DMA,
            pltpu.SemaphoreType.DMA,
        ],
    )
    def prop(h_hbm, src_hbm, dst_hbm, zeros_hbm, out0, out1,
             sidx_all, didx0, didx1, didxt, rows, rowst, acc,
             gsem0, gsem1, isem0, isem1):
        c = lax.axis_index("c")
        s = lax.axis_index("s")
        wid = s * NC + c
        tile_rows_copy(zeros_hbm, acc, s)
        base = wid * epw
        pltpu.sync_copy(src_hbm.at[pl.ds(base, epw)], sidx_all)
        plsc.subcore_barrier()

        didxs = (didx0, didx1)
        isems = (isem0, isem1)
        gsems = (gsem0, gsem1)

        def gather(j, p, sem):
            return pltpu.async_copy(
                h_hbm.at[sidx_all.at[pl.ds(j * B, B)]], rows.at[p], sem)

        def gather_wait(p, sem):
            pltpu.make_async_copy(
                h_hbm.at[sidx_all.at[pl.ds(0, B)]], rows.at[p], sem).wait()

        def idx_load(j, p):
            return pltpu.async_copy(
                dst_hbm.at[pl.ds(pl.multiple_of(base + j * B, 8), B)],
                didxs[p], isems[p])

        def idx_wait(p):
            pltpu.make_async_copy(dst_hbm.at[pl.ds(base, B)],
                                  didxs[p], isems[p]).wait()

        if nfull > 0:
            pltpu.sync_copy(dst_hbm.at[pl.ds(base, B)], didx0)
            gather(0, 0, gsem0)
            if nfull > 1:
                pltpu.sync_copy(dst_hbm.at[pl.ds(base + B, B)], didx1)

        def half(t, j, p):
            gather_wait(p, gsems[p])

            @pl.when(j + 1 < nfull)
            def _():
                gather(j + 1, 1 - p, gsems[1 - p])

            @pl.when(t > 0)
            def _():
                idx_wait(p)

            pltpu.sync_copy(rows.at[p], acc.at[didxs[p]], add=True)

            @pl.when(j + 2 < nfull)
            def _():
                idx_load(j + 2, p)

        def pair(t, carry):
            half(t, 2 * t, 0)
            half(t, 2 * t + 1, 1)
            return carry

        lax.fori_loop(0, pairs, pair, 0)
        if odd:
            half(pairs, nfull - 1, 0)
        if tail:
            toff = pl.multiple_of(base + nfull * B, 8)
            pltpu.sync_copy(dst_hbm.at[pl.ds(toff, tail)], didxt)
            pltpu.async_copy(
                h_hbm.at[sidx_all.at[pl.ds(nfull * B, tail)]],
                rowst, gsem0).wait()
            pltpu.sync_copy(rowst, acc.at[didxt], add=True)
        plsc.subcore_barrier()

        @pl.when(c == 0)
        def _():
            tile_rows_copy(acc, out0, s)

        @pl.when(c == 1)
        def _():
            tile_rows_copy(acc, out1, s)

    return prop


_BM = 1000


def _prelu(x, a):
    return jnp.where(x >= 0, x, a * x)


def _row_spec(d):
    return pl.BlockSpec((_BM, d), lambda i: (i, 0))


def _full_spec(r, c):
    return pl.BlockSpec((r, c), lambda i: (0, 0))


def _prep(pout0, pout1, pin0, pin1, feat):
    n, d = feat.shape

    def body(po0, po1, pi0, pi1, f, do_ref, di_ref, fdo_ref):
        dego = jnp.maximum(po0[:, :1] + po1[:, :1], 1.0)
        degi = jnp.maximum(pi0[:, :1] + pi1[:, :1], 1.0)
        do = lax.rsqrt(dego)
        di = lax.rsqrt(degi)
        do_ref[...] = do
        di_ref[...] = di
        fdo_ref[...] = f[...] * do

    return pl.pallas_call(
        body,
        grid=(n // _BM,),
        in_specs=[_row_spec(d)] * 4 + [_row_spec(d)],
        out_specs=[_row_spec(1), _row_spec(1), _row_spec(d)],
        out_shape=[
            jax.ShapeDtypeStruct((n, 1), jnp.float32),
            jax.ShapeDtypeStruct((n, 1), jnp.float32),
            jax.ShapeDtypeStruct((n, d), jnp.float32),
        ],
    )(pout0, pout1, pin0, pin1, feat)


def _mm_scale(x, w, scale):
    n, k = x.shape
    d = w.shape[1]

    def body(x_ref, w_ref, s_ref, o_ref):
        o_ref[...] = jnp.dot(x_ref[...], w_ref[...],
                             preferred_element_type=jnp.float32) * s_ref[...]

    return pl.pallas_call(
        body,
        grid=(n // _BM,),
        in_specs=[_row_spec(k), _full_spec(k, d), _row_spec(1)],
        out_specs=_row_spec(d),
        out_shape=jax.ShapeDtypeStruct((n, d), jnp.float32),
    )(x, w, scale)


def _mm_bias_prelu(x, w, b, a):
    n, k = x.shape
    d = w.shape[1]

    def body(x_ref, w_ref, b_ref, a_ref, o_ref):
        h = jnp.dot(x_ref[...], w_ref[...],
                    preferred_element_type=jnp.float32) + b_ref[...]
        o_ref[...] = _prelu(h, a_ref[0, 0])

    return pl.pallas_call(
        body,
        grid=(n // _BM,),
        in_specs=[_row_spec(k), _full_spec(k, d), _full_spec(1, d),
                  _full_spec(1, 1)],
        out_specs=_row_spec(d),
        out_shape=jax.ShapeDtypeStruct((n, d), jnp.float32),
    )(x, w, b, a)


def _gcn_step(acc0, acc1, di, b, a, w, do):
    n, d = acc0.shape

    def body(a0, a1, di_ref, b_ref, al_ref, w_ref, do_ref, x_ref, y_ref):
        x = _prelu((a0[...] + a1[...]) * di_ref[...] + b_ref[...], al_ref[0, 0])
        x_ref[...] = x
        y_ref[...] = jnp.dot(x, w_ref[...],
                             preferred_element_type=jnp.float32) * do_ref[...]

    return pl.pallas_call(
        body,
        grid=(n // _BM,),
        in_specs=[_row_spec(d), _row_spec(d), _row_spec(1), _full_spec(1, d),
                  _full_spec(1, 1), _full_spec(d, d), _row_spec(1)],
        out_specs=[_row_spec(d), _row_spec(d)],
        out_shape=[jax.ShapeDtypeStruct((n, d), jnp.float32),
                   jax.ShapeDtypeStruct((n, d), jnp.float32)],
    )(acc0, acc1, di, b, a, w, do)


def _gcn_last(acc0, acc1, di, b, a):
    n, d = acc0.shape

    def body(a0, a1, di_ref, b_ref, al_ref, x_ref):
        x_ref[...] = _prelu((a0[...] + a1[...]) * di_ref[...] + b_ref[...],
                            al_ref[0, 0])

    return pl.pallas_call(
        body,
        grid=(n // _BM,),
        in_specs=[_row_spec(d), _row_spec(d), _row_spec(1), _full_spec(1, d),
                  _full_spec(1, 1)],
        out_specs=_row_spec(d),
        out_shape=jax.ShapeDtypeStruct((n, d), jnp.float32),
    )(acc0, acc1, di, b, a)


def _appnp_step(acc0, acc1, di, h0, do, last):
    n, d = acc0.shape

    def body(a0, a1, di_ref, h0_ref, do_ref, o_ref):
        h = (1.0 - ALPHA) * (a0[...] + a1[...]) * di_ref[...] \
            + ALPHA * h0_ref[...]
        o_ref[...] = h if last else h * do_ref[...]

    return pl.pallas_call(
        body,
        grid=(n // _BM,),
        in_specs=[_row_spec(d), _row_spec(d), _row_spec(1), _row_spec(d),
                  _row_spec(1)],
        out_specs=_row_spec(d),
        out_shape=jax.ShapeDtypeStruct((n, d), jnp.float32),
    )(acc0, acc1, di, h0, do)


def _segsum(x, gid2, g):
    n, d = x.shape

    def body(x_ref, gid_ref, o_ref):
        @pl.when(pl.program_id(0) == 0)
        def _():
            o_ref[...] = jnp.zeros_like(o_ref)

        cols = lax.broadcasted_iota(jnp.int32, (_BM, g), 1)
        onehot = (gid_ref[...] == cols).astype(jnp.float32)
        o_ref[...] += lax.dot_general(
            onehot, x_ref[...], (((0,), (0,)), ((), ())),
            preferred_element_type=jnp.float32)

    return pl.pallas_call(
        body,
        grid=(n // _BM,),
        in_specs=[_row_spec(d), _row_spec(1)],
        out_specs=_full_spec(g, d),
        out_shape=jax.ShapeDtypeStruct((g, d), jnp.float32),
    )(x, gid2)


def _mlp(x, w0, b0, a0, w1, b1, a1, w2, b2, a2, ws, bs):
    n, k = x.shape
    d = w0.shape[1]
    bm = min(_BM, n)

    def body(x_ref, w0r, b0r, a0r, w1r, b1r, a1r, w2r, b2r, a2r, wsr, bsr,
             o_ref):
        xv = x_ref[...]
        h = _prelu(jnp.dot(xv, w0r[...], preferred_element_type=jnp.float32)
                   + b0r[...], a0r[0, 0])
        h = _prelu(jnp.dot(h, w1r[...], preferred_element_type=jnp.float32)
                   + b1r[...], a1r[0, 0])
        h = _prelu(jnp.dot(h, w2r[...], preferred_element_type=jnp.float32)
                   + b2r[...], a2r[0, 0])
        o_ref[...] = h + jnp.dot(xv, wsr[...],
                                 preferred_element_type=jnp.float32) + bsr[...]

    row = pl.BlockSpec((bm, k), lambda i: (i, 0))
    rowo = pl.BlockSpec((bm, d), lambda i: (i, 0))
    return pl.pallas_call(
        body,
        grid=(n // bm,),
        in_specs=[row,
                  _full_spec(k, d), _full_spec(1, d), _full_spec(1, 1),
                  _full_spec(d, d), _full_spec(1, d), _full_spec(1, 1),
                  _full_spec(d, d), _full_spec(1, d), _full_spec(1, 1),
                  _full_spec(k, d), _full_spec(1, d)],
        out_specs=rowo,
        out_shape=jax.ShapeDtypeStruct((n, d), jnp.float32),
    )(x, w0, b0, a0, w1, b1, a1, w2, b2, a2, ws, bs)


def _loss_sums(l_enc, g_enc, gid2):
    n, d = l_enc.shape
    g = g_enc.shape[0]
    log2 = 0.6931471805599453

    def body(l_ref, g_ref, gid_ref, p_ref, n_ref):
        @pl.when(pl.program_id(0) == 0)
        def _():
            p_ref[...] = jnp.zeros_like(p_ref)
            n_ref[...] = jnp.zeros_like(n_ref)

        res = lax.dot_general(l_ref[...], g_ref[...],
                              (((1,), (1,)), ((), ())),
                              preferred_element_type=jnp.float32)
        cols = lax.broadcasted_iota(jnp.int32, (_BM, g), 1)
        mask = (gid_ref[...] == cols).astype(jnp.float32)
        sp = jax.nn.softplus(-res)
        p_ref[...] += jnp.sum(mask * (log2 - sp)).reshape(1, 1)
        n_ref[...] += jnp.sum((1.0 - mask) * (sp + res - log2)).reshape(1, 1)

    return pl.pallas_call(
        body,
        grid=(n // _BM,),
        in_specs=[_row_spec(d), _full_spec(g, d), _row_spec(1)],
        out_specs=[_full_spec(1, 1), _full_spec(1, 1)],
        out_shape=[jax.ShapeDtypeStruct((1, 1), jnp.float32),
                   jax.ShapeDtypeStruct((1, 1), jnp.float32)],
    )(l_enc, g_enc, gid2)


def kernel(feat, params, edge_index, graph_id):
    n, d = feat.shape
    e = edge_index.shape[1]
    g = 128
    src = edge_index[0]
    dst = edge_index[1]
    gid2 = graph_id[:, None]

    zeros_nd = jnp.zeros((n, d), jnp.float32)
    ones_nd = jnp.ones((n, d), jnp.float32)

    prop = _make_propagate(n, d, e)

    di0, di1 = prop(ones_nd, src, dst, zeros_nd)
    do0, do1 = prop(ones_nd, dst, src, zeros_nd)
    do, di, feat_do = _prep(do0, do1, di0, di1, feat)

    def r2(v):
        return jnp.reshape(v, (1, 1))

    def gcn(x0, ws, bs, a, nlayers):
        segs = []
        y = _mm_scale(x0, ws[0], do)
        for l in range(nlayers):
            a0, a1 = prop(y, src, dst, zeros_nd)
            if l + 1 < nlayers:
                x, y = _gcn_step(a0, a1, di, bs[l][None, :], r2(a),
                                 ws[l + 1], do)
            else:
                x = _gcn_last(a0, a1, di, bs[l][None, :], r2(a))
            segs.append(_segsum(x, gid2, g))
        return x, segs

    p = params

    ws1, bs1, al1 = p["enc1_W"], p["enc1_b"], p["enc1_a"]
    segs1 = []
    y = _mm_scale(feat, ws1[0], do)
    hdo = feat_do
    xg = None
    for k in range(KPROP):
        pa0, pa1 = prop(hdo, src, dst, zeros_nd)
        if k < 4:
            ga0, ga1 = prop(y, src, dst, zeros_nd)
            if k < 3:
                xg, y = _gcn_step(ga0, ga1, di, bs1[k][None, :], r2(al1),
                                  ws1[k + 1], do)
            else:
                xg = _gcn_last(ga0, ga1, di, bs1[3][None, :], r2(al1))
            segs1.append(_segsum(xg, gid2, g))
        hdo = _appnp_step(pa0, pa1, di, feat, do, last=(k == KPROP - 1))
    local_v1 = xg
    global_v1 = jnp.concatenate(segs1, axis=-1)

    out = _mm_bias_prelu(hdo, p["ppnp_W"], p["ppnp_b"][None, :],
                         r2(p["ppnp_a"]))
    out_global = _segsum(out, gid2, g)

    local_v2, segs2 = gcn(out, p["enc2_W"], p["enc2_b"], p["enc2_a"], 3)
    global_v2 = jnp.concatenate([out_global] + segs2, axis=-1)

    lm = p["local_mlp"]
    gm = p["global_mlp"]

    def run_mlp(x, m):
        return _mlp(x, m["W0"], m["b0"][None, :], r2(m["a0"]),
                    m["W1"], m["b1"][None, :], r2(m["a1"]),
                    m["W2"], m["b2"][None, :], r2(m["a2"]),
                    m["Ws"], m["bs"][None, :])

    local_v1 = run_mlp(local_v1, lm)
    local_v2 = run_mlp(local_v2, lm)
    global_v1 = run_mlp(global_v1, gm)
    global_v2 = run_mlp(global_v2, gm)

    def lgl(l_enc, g_enc):
        ps, ns = _loss_sums(l_enc, g_enc, gid2)
        e_pos = ps[0, 0] / n
        e_neg = ns[0, 0] / (n * (g - 1))
        return e_neg - e_pos

    return lgl(local_v1, global_v2) + lgl(local_v2, global_v1)


# speedup vs baseline: 1.1805x; 1.1805x over previous
import functools

import jax
import jax.numpy as jnp
from jax import lax
from jax.experimental import pallas as pl
from jax.experimental.pallas import tpu as pltpu
from jax.experimental.pallas import tpu_sc as plsc

NC = 2
NS = 16
NW = NC * NS

ALPHA = 0.2
KPROP = 20


@functools.lru_cache(maxsize=None)
def _make_propagate(n, d, e, dtype=jnp.float32):
    epw = e // NW
    B = 128
    nfull = epw // B
    tail = epw - nfull * B
    pairs = nfull // 2
    odd = nfull - pairs * 2
    rows_a = (n // NS) // 8 * 8
    rem = n - rows_a * NS
    mesh = plsc.VectorSubcoreMesh(core_axis_name="c", subcore_axis_name="s")

    def tile_rows_copy(src, dst, s):
        pltpu.sync_copy(src.at[pl.ds(s * rows_a, rows_a)],
                        dst.at[pl.ds(s * rows_a, rows_a)])
        if rem:
            @pl.when(s == NS - 1)
            def _():
                pltpu.sync_copy(src.at[pl.ds(rows_a * NS, rem)],
                                dst.at[pl.ds(rows_a * NS, rem)])

    @functools.partial(
        pl.kernel,
        out_type=[jax.ShapeDtypeStruct((n, d), dtype) for _ in range(NC)],
        mesh=mesh,
        scratch_types=[
            pltpu.VMEM((epw,), jnp.int32),
            pltpu.VMEM((B,), jnp.int32),
            pltpu.VMEM((B,), jnp.int32),
            pltpu.VMEM((tail if tail else 8,), jnp.int32),
            pltpu.VMEM((2, B, d), dtype),
            pltpu.VMEM((tail if tail else 8, d), dtype),
            pltpu.VMEM_SHARED((n, d), dtype),
            pltpu.SemaphoreType.DMA,
            pltpu.SemaphoreType.DMA,
            pltpu.SemaphoreType.DMA,
            pltpu.SemaphoreType.DMA,
        ],
    )
    def prop(h_hbm, src_hbm, dst_hbm, zeros_hbm, out0, out1,
             sidx_all, didx0, didx1, didxt, rows, rowst, acc,
             gsem0, gsem1, isem0, isem1):
        c = lax.axis_index("c")
        s = lax.axis_index("s")
        wid = s * NC + c
        tile_rows_copy(zeros_hbm, acc, s)
        base = wid * epw
        pltpu.sync_copy(src_hbm.at[pl.ds(base, epw)], sidx_all)
        plsc.subcore_barrier()

        didxs = (didx0, didx1)
        isems = (isem0, isem1)
        gsems = (gsem0, gsem1)

        def gather(j, p, sem):
            return pltpu.async_copy(
                h_hbm.at[sidx_all.at[pl.ds(j * B, B)]], rows.at[p], sem)

        def gather_wait(p, sem):
            pltpu.make_async_copy(
                h_hbm.at[sidx_all.at[pl.ds(0, B)]], rows.at[p], sem).wait()

        def idx_load(j, p):
            return pltpu.async_copy(
                dst_hbm.at[pl.ds(pl.multiple_of(base + j * B, 8), B)],
                didxs[p], isems[p])

        def idx_wait(p):
            pltpu.make_async_copy(dst_hbm.at[pl.ds(base, B)],
                                  didxs[p], isems[p]).wait()

        if nfull > 0:
            pltpu.sync_copy(dst_hbm.at[pl.ds(base, B)], didx0)
            gather(0, 0, gsem0)
            if nfull > 1:
                pltpu.sync_copy(dst_hbm.at[pl.ds(base + B, B)], didx1)

        def half(t, j, p):
            @pl.when(j + 1 < nfull)
            def _():
                gather(j + 1, 1 - p, gsems[1 - p])

            gather_wait(p, gsems[p])

            @pl.when(t > 0)
            def _():
                idx_wait(p)

            pltpu.sync_copy(rows.at[p], acc.at[didxs[p]], add=True)

            @pl.when(j + 2 < nfull)
            def _():
                idx_load(j + 2, p)

        def pair(t, carry):
            half(t, 2 * t, 0)
            half(t, 2 * t + 1, 1)
            return carry

        lax.fori_loop(0, pairs, pair, 0)
        if odd:
            half(pairs, nfull - 1, 0)
        if tail:
            toff = pl.multiple_of(base + nfull * B, 8)
            pltpu.sync_copy(dst_hbm.at[pl.ds(toff, tail)], didxt)
            pltpu.async_copy(
                h_hbm.at[sidx_all.at[pl.ds(nfull * B, tail)]],
                rowst, gsem0).wait()
            pltpu.sync_copy(rowst, acc.at[didxt], add=True)
        plsc.subcore_barrier()

        @pl.when(c == 0)
        def _():
            tile_rows_copy(acc, out0, s)

        @pl.when(c == 1)
        def _():
            tile_rows_copy(acc, out1, s)

    return prop


_BM = 1000


def _prelu(x, a):
    return jnp.where(x >= 0, x, a * x)


def _row_spec(d):
    return pl.BlockSpec((_BM, d), lambda i: (i, 0))


def _full_spec(r, c):
    return pl.BlockSpec((r, c), lambda i: (0, 0))


def _prep(pout0, pout1, pin0, pin1, feat):
    n, d = feat.shape

    def body(po0, po1, pi0, pi1, f, do_ref, di_ref, fdo_ref):
        dego = jnp.maximum(po0[:, :1] + po1[:, :1], 1.0)
        degi = jnp.maximum(pi0[:, :1] + pi1[:, :1], 1.0)
        do = lax.rsqrt(dego)
        di = lax.rsqrt(degi)
        do_ref[...] = do
        di_ref[...] = di
        fdo_ref[...] = f[...] * do

    return pl.pallas_call(
        body,
        grid=(n // _BM,),
        in_specs=[_row_spec(d)] * 4 + [_row_spec(d)],
        out_specs=[_row_spec(1), _row_spec(1), _row_spec(d)],
        out_shape=[
            jax.ShapeDtypeStruct((n, 1), jnp.float32),
            jax.ShapeDtypeStruct((n, 1), jnp.float32),
            jax.ShapeDtypeStruct((n, d), jnp.float32),
        ],
    )(pout0, pout1, pin0, pin1, feat)


def _mm_scale(x, w, scale):
    n, k = x.shape
    d = w.shape[1]

    def body(x_ref, w_ref, s_ref, o_ref):
        o_ref[...] = jnp.dot(x_ref[...], w_ref[...],
                             preferred_element_type=jnp.float32) * s_ref[...]

    return pl.pallas_call(
        body,
        grid=(n // _BM,),
        in_specs=[_row_spec(k), _full_spec(k, d), _row_spec(1)],
        out_specs=_row_spec(d),
        out_shape=jax.ShapeDtypeStruct((n, d), jnp.float32),
    )(x, w, scale)


def _mm_bias_prelu(x, w, b, a):
    n, k = x.shape
    d = w.shape[1]

    def body(x_ref, w_ref, b_ref, a_ref, o_ref):
        h = jnp.dot(x_ref[...], w_ref[...],
                    preferred_element_type=jnp.float32) + b_ref[...]
        o_ref[...] = _prelu(h, a_ref[0, 0])

    return pl.pallas_call(
        body,
        grid=(n // _BM,),
        in_specs=[_row_spec(k), _full_spec(k, d), _full_spec(1, d),
                  _full_spec(1, 1)],
        out_specs=_row_spec(d),
        out_shape=jax.ShapeDtypeStruct((n, d), jnp.float32),
    )(x, w, b, a)


def _gcn_step(acc0, acc1, di, b, a, w, do):
    n, d = acc0.shape

    def body(a0, a1, di_ref, b_ref, al_ref, w_ref, do_ref, x_ref, y_ref):
        x = _prelu((a0[...] + a1[...]) * di_ref[...] + b_ref[...], al_ref[0, 0])
        x_ref[...] = x
        y_ref[...] = jnp.dot(x, w_ref[...],
                             preferred_element_type=jnp.float32) * do_ref[...]

    return pl.pallas_call(
        body,
        grid=(n // _BM,),
        in_specs=[_row_spec(d), _row_spec(d), _row_spec(1), _full_spec(1, d),
                  _full_spec(1, 1), _full_spec(d, d), _row_spec(1)],
        out_specs=[_row_spec(d), _row_spec(d)],
        out_shape=[jax.ShapeDtypeStruct((n, d), jnp.float32),
                   jax.ShapeDtypeStruct((n, d), jnp.float32)],
    )(acc0, acc1, di, b, a, w, do)


def _gcn_last(acc0, acc1, di, b, a):
    n, d = acc0.shape

    def body(a0, a1, di_ref, b_ref, al_ref, x_ref):
        x_ref[...] = _prelu((a0[...] + a1[...]) * di_ref[...] + b_ref[...],
                            al_ref[0, 0])

    return pl.pallas_call(
        body,
        grid=(n // _BM,),
        in_specs=[_row_spec(d), _row_spec(d), _row_spec(1), _full_spec(1, d),
                  _full_spec(1, 1)],
        out_specs=_row_spec(d),
        out_shape=jax.ShapeDtypeStruct((n, d), jnp.float32),
    )(acc0, acc1, di, b, a)


def _appnp_step(acc0, acc1, di, h0, do, last):
    n, d = acc0.shape

    def body(a0, a1, di_ref, h0_ref, do_ref, o_ref):
        h = (1.0 - ALPHA) * (a0[...] + a1[...]) * di_ref[...] \
            + ALPHA * h0_ref[...]
        o_ref[...] = h if last else h * do_ref[...]

    return pl.pallas_call(
        body,
        grid=(n // _BM,),
        in_specs=[_row_spec(d), _row_spec(d), _row_spec(1), _row_spec(d),
                  _row_spec(1)],
        out_specs=_row_spec(d),
        out_shape=jax.ShapeDtypeStruct((n, d), jnp.float32),
    )(acc0, acc1, di, h0, do)


def _segsum(x, gid2, g):
    n, d = x.shape

    def body(x_ref, gid_ref, o_ref):
        @pl.when(pl.program_id(0) == 0)
        def _():
            o_ref[...] = jnp.zeros_like(o_ref)

        cols = lax.broadcasted_iota(jnp.int32, (_BM, g), 1)
        onehot = (gid_ref[...] == cols).astype(jnp.float32)
        o_ref[...] += lax.dot_general(
            onehot, x_ref[...], (((0,), (0,)), ((), ())),
            preferred_element_type=jnp.float32)

    return pl.pallas_call(
        body,
        grid=(n // _BM,),
        in_specs=[_row_spec(d), _row_spec(1)],
        out_specs=_full_spec(g, d),
        out_shape=jax.ShapeDtypeStruct((g, d), jnp.float32),
    )(x, gid2)


def _mlp(x, w0, b0, a0, w1, b1, a1, w2, b2, a2, ws, bs):
    n, k = x.shape
    d = w0.shape[1]
    bm = min(_BM, n)

    def body(x_ref, w0r, b0r, a0r, w1r, b1r, a1r, w2r, b2r, a2r, wsr, bsr,
             o_ref):
        xv = x_ref[...]
        h = _prelu(jnp.dot(xv, w0r[...], preferred_element_type=jnp.float32)
                   + b0r[...], a0r[0, 0])
        h = _prelu(jnp.dot(h, w1r[...], preferred_element_type=jnp.float32)
                   + b1r[...], a1r[0, 0])
        h = _prelu(jnp.dot(h, w2r[...], preferred_element_type=jnp.float32)
                   + b2r[...], a2r[0, 0])
        o_ref[...] = h + jnp.dot(xv, wsr[...],
                                 preferred_element_type=jnp.float32) + bsr[...]

    row = pl.BlockSpec((bm, k), lambda i: (i, 0))
    rowo = pl.BlockSpec((bm, d), lambda i: (i, 0))
    return pl.pallas_call(
        body,
        grid=(n // bm,),
        in_specs=[row,
                  _full_spec(k, d), _full_spec(1, d), _full_spec(1, 1),
                  _full_spec(d, d), _full_spec(1, d), _full_spec(1, 1),
                  _full_spec(d, d), _full_spec(1, d), _full_spec(1, 1),
                  _full_spec(k, d), _full_spec(1, d)],
        out_specs=rowo,
        out_shape=jax.ShapeDtypeStruct((n, d), jnp.float32),
    )(x, w0, b0, a0, w1, b1, a1, w2, b2, a2, ws, bs)


def _loss_sums(l_enc, g_enc, gid2):
    n, d = l_enc.shape
    g = g_enc.shape[0]
    log2 = 0.6931471805599453

    def body(l_ref, g_ref, gid_ref, p_ref, n_ref):
        @pl.when(pl.program_id(0) == 0)
        def _():
            p_ref[...] = jnp.zeros_like(p_ref)
            n_ref[...] = jnp.zeros_like(n_ref)

        res = lax.dot_general(l_ref[...], g_ref[...],
                              (((1,), (1,)), ((), ())),
                              preferred_element_type=jnp.float32)
        cols = lax.broadcasted_iota(jnp.int32, (_BM, g), 1)
        mask = (gid_ref[...] == cols).astype(jnp.float32)
        sp = jax.nn.softplus(-res)
        p_ref[...] += jnp.sum(mask * (log2 - sp)).reshape(1, 1)
        n_ref[...] += jnp.sum((1.0 - mask) * (sp + res - log2)).reshape(1, 1)

    return pl.pallas_call(
        body,
        grid=(n // _BM,),
        in_specs=[_row_spec(d), _full_spec(g, d), _row_spec(1)],
        out_specs=[_full_spec(1, 1), _full_spec(1, 1)],
        out_shape=[jax.ShapeDtypeStruct((1, 1), jnp.float32),
                   jax.ShapeDtypeStruct((1, 1), jnp.float32)],
    )(l_enc, g_enc, gid2)


def kernel(feat, params, edge_index, graph_id):
    n, d = feat.shape
    e = edge_index.shape[1]
    g = 128
    src = edge_index[0]
    dst = edge_index[1]
    gid2 = graph_id[:, None]

    zeros_nd = jnp.zeros((n, d), jnp.float32)
    ones_nd = jnp.ones((n, d), jnp.float32)

    prop = _make_propagate(n, d, e)

    di0, di1 = prop(ones_nd, src, dst, zeros_nd)
    do0, do1 = prop(ones_nd, dst, src, zeros_nd)
    do, di, feat_do = _prep(do0, do1, di0, di1, feat)

    def r2(v):
        return jnp.reshape(v, (1, 1))

    def gcn(x0, ws, bs, a, nlayers):
        segs = []
        y = _mm_scale(x0, ws[0], do)
        for l in range(nlayers):
            a0, a1 = prop(y, src, dst, zeros_nd)
            if l + 1 < nlayers:
                x, y = _gcn_step(a0, a1, di, bs[l][None, :], r2(a),
                                 ws[l + 1], do)
            else:
                x = _gcn_last(a0, a1, di, bs[l][None, :], r2(a))
            segs.append(_segsum(x, gid2, g))
        return x, segs

    p = params

    ws1, bs1, al1 = p["enc1_W"], p["enc1_b"], p["enc1_a"]
    segs1 = []
    y = _mm_scale(feat, ws1[0], do)
    hdo = feat_do
    xg = None
    for k in range(KPROP):
        pa0, pa1 = prop(hdo, src, dst, zeros_nd)
        if k < 4:
            ga0, ga1 = prop(y, src, dst, zeros_nd)
            if k < 3:
                xg, y = _gcn_step(ga0, ga1, di, bs1[k][None, :], r2(al1),
                                  ws1[k + 1], do)
            else:
                xg = _gcn_last(ga0, ga1, di, bs1[3][None, :], r2(al1))
            segs1.append(_segsum(xg, gid2, g))
        hdo = _appnp_step(pa0, pa1, di, feat, do, last=(k == KPROP - 1))
    local_v1 = xg
    global_v1 = jnp.concatenate(segs1, axis=-1)

    out = _mm_bias_prelu(hdo, p["ppnp_W"], p["ppnp_b"][None, :],
                         r2(p["ppnp_a"]))
    out_global = _segsum(out, gid2, g)

    local_v2, segs2 = gcn(out, p["enc2_W"], p["enc2_b"], p["enc2_a"], 3)
    global_v2 = jnp.concatenate([out_global] + segs2, axis=-1)

    lm = p["local_mlp"]
    gm = p["global_mlp"]

    def run_mlp(x, m):
        return _mlp(x, m["W0"], m["b0"][None, :], r2(m["a0"]),
                    m["W1"], m["b1"][None, :], r2(m["a1"]),
                    m["W2"], m["b2"][None, :], r2(m["a2"]),
                    m["Ws"], m["bs"][None, :])

    local_v1 = run_mlp(local_v1, lm)
    local_v2 = run_mlp(local_v2, lm)
    global_v1 = run_mlp(global_v1, gm)
    global_v2 = run_mlp(global_v2, gm)

    def lgl(l_enc, g_enc):
        ps, ns = _loss_sums(l_enc, g_enc, gid2)
        e_pos = ps[0, 0] / n
        e_neg = ns[0, 0] / (n * (g - 1))
        return e_neg - e_pos

    return lgl(local_v1, global_v2) + lgl(local_v2, global_v1)


# speedup vs baseline: 1.1838x; 1.0028x over previous
import functools

import jax
import jax.numpy as jnp
from jax import lax
from jax.experimental import pallas as pl
from jax.experimental.pallas import tpu as pltpu
from jax.experimental.pallas import tpu_sc as plsc

NC = 2
NS = 16
NW = NC * NS

ALPHA = 0.2
KPROP = 20


@functools.lru_cache(maxsize=None)
def _make_propagate(n, d, e, dtype=jnp.float32):
    epw = e // NW
    B = 128
    nfull = epw // B
    tail = epw - nfull * B
    pairs = nfull // 2
    odd = nfull - pairs * 2
    rows_a = (n // NS) // 8 * 8
    rem = n - rows_a * NS
    mesh = plsc.VectorSubcoreMesh(core_axis_name="c", subcore_axis_name="s")

    def tile_rows_copy(src, dst, s):
        pltpu.sync_copy(src.at[pl.ds(s * rows_a, rows_a)],
                        dst.at[pl.ds(s * rows_a, rows_a)])
        if rem:
            @pl.when(s == NS - 1)
            def _():
                pltpu.sync_copy(src.at[pl.ds(rows_a * NS, rem)],
                                dst.at[pl.ds(rows_a * NS, rem)])

    @functools.partial(
        pl.kernel,
        out_type=[jax.ShapeDtypeStruct((n, d), dtype) for _ in range(NC)],
        mesh=mesh,
        scratch_types=[
            pltpu.VMEM((epw,), jnp.int32),
            pltpu.VMEM((B,), jnp.int32),
            pltpu.VMEM((B,), jnp.int32),
            pltpu.VMEM((tail if tail else 8,), jnp.int32),
            pltpu.VMEM((2, B, d), dtype),
            pltpu.VMEM((tail if tail else 8, d), dtype),
            pltpu.VMEM_SHARED((n, d), dtype),
            pltpu.SemaphoreType.DMA,
            pltpu.SemaphoreType.DMA,
            pltpu.SemaphoreType.DMA,
            pltpu.SemaphoreType.DMA,
        ],
    )
    def prop(h_hbm, src_hbm, dst_hbm, zeros_hbm, out0, out1,
             sidx_all, didx0, didx1, didxt, rows, rowst, acc,
             gsem0, gsem1, isem0, isem1):
        c = lax.axis_index("c")
        s = lax.axis_index("s")
        wid = s * NC + c
        tile_rows_copy(zeros_hbm, acc, s)
        base = wid * epw
        pltpu.sync_copy(src_hbm.at[pl.ds(base, epw)], sidx_all)
        plsc.subcore_barrier()

        didxs = (didx0, didx1)
        isems = (isem0, isem1)
        gsems = (gsem0, gsem1)

        def gather(j, p, sem):
            return pltpu.async_copy(
                h_hbm.at[sidx_all.at[pl.ds(j * B, B)]], rows.at[p], sem)

        def gather_wait(p, sem):
            pltpu.make_async_copy(
                h_hbm.at[sidx_all.at[pl.ds(0, B)]], rows.at[p], sem).wait()

        def idx_load(j, p):
            return pltpu.async_copy(
                dst_hbm.at[pl.ds(pl.multiple_of(base + j * B, 8), B)],
                didxs[p], isems[p])

        def idx_wait(p):
            pltpu.make_async_copy(dst_hbm.at[pl.ds(base, B)],
                                  didxs[p], isems[p]).wait()

        if nfull > 0:
            pltpu.sync_copy(dst_hbm.at[pl.ds(base, B)], didx0)
            gather(0, 0, gsem0)
            if nfull > 1:
                pltpu.sync_copy(dst_hbm.at[pl.ds(base + B, B)], didx1)

        def half(t, j, p):
            @pl.when(j + 1 < nfull)
            def _():
                gather(j + 1, 1 - p, gsems[1 - p])

            gather_wait(p, gsems[p])

            @pl.when(t > 0)
            def _():
                idx_wait(p)

            pltpu.sync_copy(rows.at[p], acc.at[didxs[p]], add=True)

            @pl.when(j + 2 < nfull)
            def _():
                idx_load(j + 2, p)

        def pair(t, carry):
            half(t, 2 * t, 0)
            half(t, 2 * t + 1, 1)
            return carry

        lax.fori_loop(0, pairs, pair, 0)
        if odd:
            half(pairs, nfull - 1, 0)
        if tail:
            toff = pl.multiple_of(base + nfull * B, 8)
            pltpu.sync_copy(dst_hbm.at[pl.ds(toff, tail)], didxt)
            pltpu.async_copy(
                h_hbm.at[sidx_all.at[pl.ds(nfull * B, tail)]],
                rowst, gsem0).wait()
            pltpu.sync_copy(rowst, acc.at[didxt], add=True)
        plsc.subcore_barrier()

        @pl.when(c == 0)
        def _():
            tile_rows_copy(acc, out0, s)

        @pl.when(c == 1)
        def _():
            tile_rows_copy(acc, out1, s)

    return prop


_BM = 1000


def _prelu(x, a):
    return jnp.where(x >= 0, x, a * x)


def _row_spec(d):
    return pl.BlockSpec((_BM, d), lambda i: (i, 0))


def _full_spec(r, c):
    return pl.BlockSpec((r, c), lambda i: (0, 0))


def _prep(pout0, pout1, pin0, pin1, feat):
    n, d = feat.shape

    def body(po0, po1, pi0, pi1, f, do_ref, di_ref, fdo_ref):
        dego = jnp.maximum(po0[:, :1] + po1[:, :1], 1.0)
        degi = jnp.maximum(pi0[:, :1] + pi1[:, :1], 1.0)
        do = lax.rsqrt(dego)
        di = lax.rsqrt(degi)
        do_ref[...] = do
        di_ref[...] = di
        fdo_ref[...] = f[...] * do

    return pl.pallas_call(
        body,
        grid=(n // _BM,),
        in_specs=[_row_spec(d)] * 4 + [_row_spec(d)],
        out_specs=[_row_spec(1), _row_spec(1), _row_spec(d)],
        out_shape=[
            jax.ShapeDtypeStruct((n, 1), jnp.float32),
            jax.ShapeDtypeStruct((n, 1), jnp.float32),
            jax.ShapeDtypeStruct((n, d), jnp.float32),
        ],
    )(pout0, pout1, pin0, pin1, feat)


def _mm_scale(x, w, scale):
    n, k = x.shape
    d = w.shape[1]

    def body(x_ref, w_ref, s_ref, o_ref):
        o_ref[...] = jnp.dot(x_ref[...], w_ref[...],
                             preferred_element_type=jnp.float32) * s_ref[...]

    return pl.pallas_call(
        body,
        grid=(n // _BM,),
        in_specs=[_row_spec(k), _full_spec(k, d), _row_spec(1)],
        out_specs=_row_spec(d),
        out_shape=jax.ShapeDtypeStruct((n, d), jnp.float32),
    )(x, w, scale)


def _mm_bias_prelu(x, w, b, a, gid2, g):
    n, k = x.shape
    d = w.shape[1]

    def body(x_ref, w_ref, b_ref, a_ref, gid_ref, o_ref, hg_ref):
        h = jnp.dot(x_ref[...], w_ref[...],
                    preferred_element_type=jnp.float32) + b_ref[...]
        o = _prelu(h, a_ref[0, 0])
        o_ref[...] = o
        _seg_accum(hg_ref, o, gid_ref, g, pl.program_id(0))

    return pl.pallas_call(
        body,
        grid=(n // _BM,),
        in_specs=[_row_spec(k), _full_spec(k, d), _full_spec(1, d),
                  _full_spec(1, 1), _row_spec(1)],
        out_specs=[_row_spec(d), _full_spec(g, d)],
        out_shape=[jax.ShapeDtypeStruct((n, d), jnp.float32),
                   jax.ShapeDtypeStruct((g, d), jnp.float32)],
    )(x, w, b, a, gid2)


def _seg_accum(hg_ref, x, gid_ref, g, pid):
    @pl.when(pid == 0)
    def _():
        hg_ref[...] = jnp.zeros_like(hg_ref)

    cols = lax.broadcasted_iota(jnp.int32, (x.shape[0], g), 1)
    onehot = (gid_ref[...] == cols).astype(jnp.float32)
    hg_ref[...] += lax.dot_general(onehot, x, (((0,), (0,)), ((), ())),
                                   preferred_element_type=jnp.float32)


def _gcn_step(acc0, acc1, di, b, a, w, do, gid2, g):
    n, d = acc0.shape

    def body(a0, a1, di_ref, b_ref, al_ref, w_ref, do_ref, gid_ref,
             x_ref, y_ref, hg_ref):
        x = _prelu((a0[...] + a1[...]) * di_ref[...] + b_ref[...], al_ref[0, 0])
        x_ref[...] = x
        y_ref[...] = jnp.dot(x, w_ref[...],
                             preferred_element_type=jnp.float32) * do_ref[...]
        _seg_accum(hg_ref, x, gid_ref, g, pl.program_id(0))

    return pl.pallas_call(
        body,
        grid=(n // _BM,),
        in_specs=[_row_spec(d), _row_spec(d), _row_spec(1), _full_spec(1, d),
                  _full_spec(1, 1), _full_spec(d, d), _row_spec(1),
                  _row_spec(1)],
        out_specs=[_row_spec(d), _row_spec(d), _full_spec(g, d)],
        out_shape=[jax.ShapeDtypeStruct((n, d), jnp.float32),
                   jax.ShapeDtypeStruct((n, d), jnp.float32),
                   jax.ShapeDtypeStruct((g, d), jnp.float32)],
    )(acc0, acc1, di, b, a, w, do, gid2)


def _gcn_last(acc0, acc1, di, b, a, gid2, g):
    n, d = acc0.shape

    def body(a0, a1, di_ref, b_ref, al_ref, gid_ref, x_ref, hg_ref):
        x = _prelu((a0[...] + a1[...]) * di_ref[...] + b_ref[...],
                   al_ref[0, 0])
        x_ref[...] = x
        _seg_accum(hg_ref, x, gid_ref, g, pl.program_id(0))

    return pl.pallas_call(
        body,
        grid=(n // _BM,),
        in_specs=[_row_spec(d), _row_spec(d), _row_spec(1), _full_spec(1, d),
                  _full_spec(1, 1), _row_spec(1)],
        out_specs=[_row_spec(d), _full_spec(g, d)],
        out_shape=[jax.ShapeDtypeStruct((n, d), jnp.float32),
                   jax.ShapeDtypeStruct((g, d), jnp.float32)],
    )(acc0, acc1, di, b, a, gid2)


def _appnp_step(acc0, acc1, di, h0, do, last):
    n, d = acc0.shape

    def body(a0, a1, di_ref, h0_ref, do_ref, o_ref):
        h = (1.0 - ALPHA) * (a0[...] + a1[...]) * di_ref[...] \
            + ALPHA * h0_ref[...]
        o_ref[...] = h if last else h * do_ref[...]

    return pl.pallas_call(
        body,
        grid=(n // _BM,),
        in_specs=[_row_spec(d), _row_spec(d), _row_spec(1), _row_spec(d),
                  _row_spec(1)],
        out_specs=_row_spec(d),
        out_shape=jax.ShapeDtypeStruct((n, d), jnp.float32),
    )(acc0, acc1, di, h0, do)


def _mlp(x, w0, b0, a0, w1, b1, a1, w2, b2, a2, ws, bs):
    n, k = x.shape
    d = w0.shape[1]
    bm = min(_BM, n)

    def body(x_ref, w0r, b0r, a0r, w1r, b1r, a1r, w2r, b2r, a2r, wsr, bsr,
             o_ref):
        xv = x_ref[...]
        h = _prelu(jnp.dot(xv, w0r[...], preferred_element_type=jnp.float32)
                   + b0r[...], a0r[0, 0])
        h = _prelu(jnp.dot(h, w1r[...], preferred_element_type=jnp.float32)
                   + b1r[...], a1r[0, 0])
        h = _prelu(jnp.dot(h, w2r[...], preferred_element_type=jnp.float32)
                   + b2r[...], a2r[0, 0])
        o_ref[...] = h + jnp.dot(xv, wsr[...],
                                 preferred_element_type=jnp.float32) + bsr[...]

    row = pl.BlockSpec((bm, k), lambda i: (i, 0))
    rowo = pl.BlockSpec((bm, d), lambda i: (i, 0))
    return pl.pallas_call(
        body,
        grid=(n // bm,),
        in_specs=[row,
                  _full_spec(k, d), _full_spec(1, d), _full_spec(1, 1),
                  _full_spec(d, d), _full_spec(1, d), _full_spec(1, 1),
                  _full_spec(d, d), _full_spec(1, d), _full_spec(1, 1),
                  _full_spec(k, d), _full_spec(1, d)],
        out_specs=rowo,
        out_shape=jax.ShapeDtypeStruct((n, d), jnp.float32),
    )(x, w0, b0, a0, w1, b1, a1, w2, b2, a2, ws, bs)


def _loss_sums(l_enc, g_enc, gid2):
    n, d = l_enc.shape
    g = g_enc.shape[0]
    log2 = 0.6931471805599453

    def body(l_ref, g_ref, gid_ref, p_ref, n_ref):
        @pl.when(pl.program_id(0) == 0)
        def _():
            p_ref[...] = jnp.zeros_like(p_ref)
            n_ref[...] = jnp.zeros_like(n_ref)

        res = lax.dot_general(l_ref[...], g_ref[...],
                              (((1,), (1,)), ((), ())),
                              preferred_element_type=jnp.float32)
        cols = lax.broadcasted_iota(jnp.int32, (_BM, g), 1)
        mask = (gid_ref[...] == cols).astype(jnp.float32)
        sp = jax.nn.softplus(-res)
        p_ref[...] += jnp.sum(mask * (log2 - sp)).reshape(1, 1)
        n_ref[...] += jnp.sum((1.0 - mask) * (sp + res - log2)).reshape(1, 1)

    return pl.pallas_call(
        body,
        grid=(n // _BM,),
        in_specs=[_row_spec(d), _full_spec(g, d), _row_spec(1)],
        out_specs=[_full_spec(1, 1), _full_spec(1, 1)],
        out_shape=[jax.ShapeDtypeStruct((1, 1), jnp.float32),
                   jax.ShapeDtypeStruct((1, 1), jnp.float32)],
    )(l_enc, g_enc, gid2)


def kernel(feat, params, edge_index, graph_id):
    n, d = feat.shape
    e = edge_index.shape[1]
    g = 128
    src = edge_index[0]
    dst = edge_index[1]
    gid2 = graph_id[:, None]

    zeros_nd = jnp.zeros((n, d), jnp.float32)
    ones_nd = jnp.ones((n, d), jnp.float32)

    prop = _make_propagate(n, d, e)

    di0, di1 = prop(ones_nd, src, dst, zeros_nd)
    do0, do1 = prop(ones_nd, dst, src, zeros_nd)
    do, di, feat_do = _prep(do0, do1, di0, di1, feat)

    def r2(v):
        return jnp.reshape(v, (1, 1))

    def gcn(x0, ws, bs, a, nlayers):
        segs = []
        y = _mm_scale(x0, ws[0], do)
        for l in range(nlayers):
            a0, a1 = prop(y, src, dst, zeros_nd)
            if l + 1 < nlayers:
                x, y, hg = _gcn_step(a0, a1, di, bs[l][None, :], r2(a),
                                     ws[l + 1], do, gid2, g)
            else:
                x, hg = _gcn_last(a0, a1, di, bs[l][None, :], r2(a), gid2, g)
            segs.append(hg)
        return x, segs

    p = params

    ws1, bs1, al1 = p["enc1_W"], p["enc1_b"], p["enc1_a"]
    segs1 = []
    y = _mm_scale(feat, ws1[0], do)
    hdo = feat_do
    xg = None
    for k in range(KPROP):
        pa0, pa1 = prop(hdo, src, dst, zeros_nd)
        if k < 4:
            ga0, ga1 = prop(y, src, dst, zeros_nd)
            if k < 3:
                xg, y, hgk = _gcn_step(ga0, ga1, di, bs1[k][None, :], r2(al1),
                                       ws1[k + 1], do, gid2, g)
            else:
                xg, hgk = _gcn_last(ga0, ga1, di, bs1[3][None, :], r2(al1),
                                    gid2, g)
            segs1.append(hgk)
        hdo = _appnp_step(pa0, pa1, di, feat, do, last=(k == KPROP - 1))
    local_v1 = xg
    global_v1 = jnp.concatenate(segs1, axis=-1)

    out, out_global = _mm_bias_prelu(hdo, p["ppnp_W"], p["ppnp_b"][None, :],
                                     r2(p["ppnp_a"]), gid2, g)

    local_v2, segs2 = gcn(out, p["enc2_W"], p["enc2_b"], p["enc2_a"], 3)
    global_v2 = jnp.concatenate([out_global] + segs2, axis=-1)

    lm = p["local_mlp"]
    gm = p["global_mlp"]

    def run_mlp(x, m):
        return _mlp(x, m["W0"], m["b0"][None, :], r2(m["a0"]),
                    m["W1"], m["b1"][None, :], r2(m["a1"]),
                    m["W2"], m["b2"][None, :], r2(m["a2"]),
                    m["Ws"], m["bs"][None, :])

    local_v1 = run_mlp(local_v1, lm)
    local_v2 = run_mlp(local_v2, lm)
    global_v1 = run_mlp(global_v1, gm)
    global_v2 = run_mlp(global_v2, gm)

    def lgl(l_enc, g_enc):
        ps, ns = _loss_sums(l_enc, g_enc, gid2)
        e_pos = ps[0, 0] / n
        e_neg = ns[0, 0] / (n * (g - 1))
        return e_neg - e_pos

    return lgl(local_v1, global_v2) + lgl(local_v2, global_v1)


# speedup vs baseline: 1.2065x; 1.0192x over previous
import functools

import jax
import jax.numpy as jnp
from jax import lax
from jax.experimental import pallas as pl
from jax.experimental.pallas import tpu as pltpu
from jax.experimental.pallas import tpu_sc as plsc

NC = 2
NS = 16
NW = NC * NS

ALPHA = 0.2
KPROP = 20


@functools.lru_cache(maxsize=None)
def _make_propagate(n, d, e, dtype=jnp.float32):
    epw = e // NW
    B = 128
    nfull = epw // B
    tail = epw - nfull * B
    pairs = nfull // 2
    odd = nfull - pairs * 2
    rows_a = (n // NS) // 8 * 8
    rem = n - rows_a * NS
    mesh = plsc.VectorSubcoreMesh(core_axis_name="c", subcore_axis_name="s")

    def tile_rows_copy(src, dst, s):
        pltpu.sync_copy(src.at[pl.ds(s * rows_a, rows_a)],
                        dst.at[pl.ds(s * rows_a, rows_a)])
        if rem:
            @pl.when(s == NS - 1)
            def _():
                pltpu.sync_copy(src.at[pl.ds(rows_a * NS, rem)],
                                dst.at[pl.ds(rows_a * NS, rem)])

    @functools.partial(
        pl.kernel,
        out_type=[jax.ShapeDtypeStruct((n, d), dtype) for _ in range(NC)],
        mesh=mesh,
        scratch_types=[
            pltpu.VMEM((epw,), jnp.int32),
            pltpu.VMEM((B,), jnp.int32),
            pltpu.VMEM((B,), jnp.int32),
            pltpu.VMEM((tail if tail else 8,), jnp.int32),
            pltpu.VMEM((2, B, d), dtype),
            pltpu.VMEM((tail if tail else 8, d), dtype),
            pltpu.VMEM_SHARED((n, d), dtype),
            pltpu.SemaphoreType.DMA,
            pltpu.SemaphoreType.DMA,
            pltpu.SemaphoreType.DMA,
            pltpu.SemaphoreType.DMA,
        ],
    )
    def prop(h_hbm, src_hbm, dst_hbm, zeros_hbm, out0, out1,
             sidx_all, didx0, didx1, didxt, rows, rowst, acc,
             gsem0, gsem1, isem0, isem1):
        c = lax.axis_index("c")
        s = lax.axis_index("s")
        wid = s * NC + c
        tile_rows_copy(zeros_hbm, acc, s)
        base = wid * epw
        pltpu.sync_copy(src_hbm.at[pl.ds(base, epw)], sidx_all)
        plsc.subcore_barrier()

        didxs = (didx0, didx1)
        isems = (isem0, isem1)
        gsems = (gsem0, gsem1)

        def gather(j, p, sem):
            return pltpu.async_copy(
                h_hbm.at[sidx_all.at[pl.ds(j * B, B)]], rows.at[p], sem)

        def gather_wait(p, sem):
            pltpu.make_async_copy(
                h_hbm.at[sidx_all.at[pl.ds(0, B)]], rows.at[p], sem).wait()

        def idx_load(j, p):
            return pltpu.async_copy(
                dst_hbm.at[pl.ds(pl.multiple_of(base + j * B, 8), B)],
                didxs[p], isems[p])

        def idx_wait(p):
            pltpu.make_async_copy(dst_hbm.at[pl.ds(base, B)],
                                  didxs[p], isems[p]).wait()

        if nfull > 0:
            pltpu.sync_copy(dst_hbm.at[pl.ds(base, B)], didx0)
            gather(0, 0, gsem0)
            if nfull > 1:
                pltpu.sync_copy(dst_hbm.at[pl.ds(base + B, B)], didx1)

        def half(t, j, p):
            @pl.when(j + 1 < nfull)
            def _():
                gather(j + 1, 1 - p, gsems[1 - p])

            gather_wait(p, gsems[p])

            @pl.when(t > 0)
            def _():
                idx_wait(p)

            pltpu.sync_copy(rows.at[p], acc.at[didxs[p]], add=True)

            @pl.when(j + 2 < nfull)
            def _():
                idx_load(j + 2, p)

        def pair(t, carry):
            half(t, 2 * t, 0)
            half(t, 2 * t + 1, 1)
            return carry

        lax.fori_loop(0, pairs, pair, 0)
        if odd:
            half(pairs, nfull - 1, 0)
        if tail:
            toff = pl.multiple_of(base + nfull * B, 8)
            pltpu.sync_copy(dst_hbm.at[pl.ds(toff, tail)], didxt)
            pltpu.async_copy(
                h_hbm.at[sidx_all.at[pl.ds(nfull * B, tail)]],
                rowst, gsem0).wait()
            pltpu.sync_copy(rowst, acc.at[didxt], add=True)
        plsc.subcore_barrier()

        @pl.when(c == 0)
        def _():
            tile_rows_copy(acc, out0, s)

        @pl.when(c == 1)
        def _():
            tile_rows_copy(acc, out1, s)

    return prop


@functools.lru_cache(maxsize=None)
def _make_count(n, d, e):
    epw = e // NW
    B = 128
    nfull = epw // B
    tail = epw - nfull * B
    pairs = nfull // 2
    odd = nfull - pairs * 2
    rows_a = (n // NS) // 8 * 8
    rem = n - rows_a * NS
    mesh = plsc.VectorSubcoreMesh(core_axis_name="c", subcore_axis_name="s")

    def tile_rows_copy(src, dst, s):
        pltpu.sync_copy(src.at[pl.ds(s * rows_a, rows_a)],
                        dst.at[pl.ds(s * rows_a, rows_a)])
        if rem:
            @pl.when(s == NS - 1)
            def _():
                pltpu.sync_copy(src.at[pl.ds(rows_a * NS, rem)],
                                dst.at[pl.ds(rows_a * NS, rem)])

    @functools.partial(
        pl.kernel,
        out_type=[jax.ShapeDtypeStruct((n, d), jnp.float32)
                  for _ in range(NC)],
        mesh=mesh,
        scratch_types=[
            pltpu.VMEM((B,), jnp.int32),
            pltpu.VMEM((B,), jnp.int32),
            pltpu.VMEM((tail if tail else 8,), jnp.int32),
            pltpu.VMEM((B, d), jnp.float32),
            pltpu.VMEM_SHARED((n, d), jnp.float32),
            pltpu.SemaphoreType.DMA,
            pltpu.SemaphoreType.DMA,
        ],
    )
    def cnt(idx_hbm, ones_hbm, zeros_hbm, out0, out1,
            didx0, didx1, didxt, ones_v, acc, isem0, isem1):
        c = lax.axis_index("c")
        s = lax.axis_index("s")
        wid = s * NC + c
        tile_rows_copy(zeros_hbm, acc, s)
        pltpu.sync_copy(ones_hbm.at[pl.ds(0, B)], ones_v)
        plsc.subcore_barrier()
        base = wid * epw

        didxs = (didx0, didx1)
        isems = (isem0, isem1)

        def idx_load(j, p):
            pltpu.async_copy(
                dst_hbm_slice(j), didxs[p], isems[p])

        def dst_hbm_slice(j):
            return idx_hbm.at[pl.ds(pl.multiple_of(base + j * B, 8), B)]

        def idx_wait(p):
            pltpu.make_async_copy(idx_hbm.at[pl.ds(base, B)],
                                  didxs[p], isems[p]).wait()

        if nfull > 0:
            pltpu.sync_copy(dst_hbm_slice(0), didx0)
            if nfull > 1:
                pltpu.sync_copy(dst_hbm_slice(1), didx1)

        def half(t, j, p):
            @pl.when(t > 0)
            def _():
                idx_wait(p)

            pltpu.sync_copy(ones_v, acc.at[didxs[p]], add=True)

            @pl.when(j + 2 < nfull)
            def _():
                idx_load(j + 2, p)

        def pair(t, carry):
            half(t, 2 * t, 0)
            half(t, 2 * t + 1, 1)
            return carry

        lax.fori_loop(0, pairs, pair, 0)
        if odd:
            half(pairs, nfull - 1, 0)
        if tail:
            toff = pl.multiple_of(base + nfull * B, 8)
            pltpu.sync_copy(idx_hbm.at[pl.ds(toff, tail)], didxt)
            pltpu.sync_copy(ones_v.at[pl.ds(0, tail)], acc.at[didxt],
                            add=True)
        plsc.subcore_barrier()

        @pl.when(c == 0)
        def _():
            tile_rows_copy(acc, out0, s)

        @pl.when(c == 1)
        def _():
            tile_rows_copy(acc, out1, s)

    return cnt


_BM = 1000


def _prelu(x, a):
    return jnp.where(x >= 0, x, a * x)


def _row_spec(d):
    return pl.BlockSpec((_BM, d), lambda i: (i, 0))


def _full_spec(r, c):
    return pl.BlockSpec((r, c), lambda i: (0, 0))


def _prep(pout0, pout1, pin0, pin1, feat):
    n, d = feat.shape

    def body(po0, po1, pi0, pi1, f, do_ref, di_ref, fdo_ref):
        dego = jnp.maximum(po0[:, :1] + po1[:, :1], 1.0)
        degi = jnp.maximum(pi0[:, :1] + pi1[:, :1], 1.0)
        do = lax.rsqrt(dego)
        di = lax.rsqrt(degi)
        do_ref[...] = do
        di_ref[...] = di
        fdo_ref[...] = f[...] * do

    return pl.pallas_call(
        body,
        grid=(n // _BM,),
        in_specs=[_row_spec(d)] * 4 + [_row_spec(d)],
        out_specs=[_row_spec(1), _row_spec(1), _row_spec(d)],
        out_shape=[
            jax.ShapeDtypeStruct((n, 1), jnp.float32),
            jax.ShapeDtypeStruct((n, 1), jnp.float32),
            jax.ShapeDtypeStruct((n, d), jnp.float32),
        ],
    )(pout0, pout1, pin0, pin1, feat)


def _mm_scale(x, w, scale):
    n, k = x.shape
    d = w.shape[1]

    def body(x_ref, w_ref, s_ref, o_ref):
        o_ref[...] = jnp.dot(x_ref[...], w_ref[...],
                             preferred_element_type=jnp.float32) * s_ref[...]

    return pl.pallas_call(
        body,
        grid=(n // _BM,),
        in_specs=[_row_spec(k), _full_spec(k, d), _row_spec(1)],
        out_specs=_row_spec(d),
        out_shape=jax.ShapeDtypeStruct((n, d), jnp.float32),
    )(x, w, scale)


def _mm_bias_prelu(x, w, b, a, gid2, g):
    n, k = x.shape
    d = w.shape[1]

    def body(x_ref, w_ref, b_ref, a_ref, gid_ref, o_ref, hg_ref):
        h = jnp.dot(x_ref[...], w_ref[...],
                    preferred_element_type=jnp.float32) + b_ref[...]
        o = _prelu(h, a_ref[0, 0])
        o_ref[...] = o
        _seg_accum(hg_ref, o, gid_ref, g, pl.program_id(0))

    return pl.pallas_call(
        body,
        grid=(n // _BM,),
        in_specs=[_row_spec(k), _full_spec(k, d), _full_spec(1, d),
                  _full_spec(1, 1), _row_spec(1)],
        out_specs=[_row_spec(d), _full_spec(g, d)],
        out_shape=[jax.ShapeDtypeStruct((n, d), jnp.float32),
                   jax.ShapeDtypeStruct((g, d), jnp.float32)],
    )(x, w, b, a, gid2)


def _seg_accum(hg_ref, x, gid_ref, g, pid):
    @pl.when(pid == 0)
    def _():
        hg_ref[...] = jnp.zeros_like(hg_ref)

    cols = lax.broadcasted_iota(jnp.int32, (x.shape[0], g), 1)
    onehot = (gid_ref[...] == cols).astype(jnp.float32)
    hg_ref[...] += lax.dot_general(onehot, x, (((0,), (0,)), ((), ())),
                                   preferred_element_type=jnp.float32)


def _gcn_step(acc0, acc1, di, b, a, w, do, gid2, g):
    n, d = acc0.shape

    def body(a0, a1, di_ref, b_ref, al_ref, w_ref, do_ref, gid_ref,
             x_ref, y_ref, hg_ref):
        x = _prelu((a0[...] + a1[...]) * di_ref[...] + b_ref[...], al_ref[0, 0])
        x_ref[...] = x
        y_ref[...] = jnp.dot(x, w_ref[...],
                             preferred_element_type=jnp.float32) * do_ref[...]
        _seg_accum(hg_ref, x, gid_ref, g, pl.program_id(0))

    return pl.pallas_call(
        body,
        grid=(n // _BM,),
        in_specs=[_row_spec(d), _row_spec(d), _row_spec(1), _full_spec(1, d),
                  _full_spec(1, 1), _full_spec(d, d), _row_spec(1),
                  _row_spec(1)],
        out_specs=[_row_spec(d), _row_spec(d), _full_spec(g, d)],
        out_shape=[jax.ShapeDtypeStruct((n, d), jnp.float32),
                   jax.ShapeDtypeStruct((n, d), jnp.float32),
                   jax.ShapeDtypeStruct((g, d), jnp.float32)],
    )(acc0, acc1, di, b, a, w, do, gid2)


def _gcn_last(acc0, acc1, di, b, a, gid2, g):
    n, d = acc0.shape

    def body(a0, a1, di_ref, b_ref, al_ref, gid_ref, x_ref, hg_ref):
        x = _prelu((a0[...] + a1[...]) * di_ref[...] + b_ref[...],
                   al_ref[0, 0])
        x_ref[...] = x
        _seg_accum(hg_ref, x, gid_ref, g, pl.program_id(0))

    return pl.pallas_call(
        body,
        grid=(n // _BM,),
        in_specs=[_row_spec(d), _row_spec(d), _row_spec(1), _full_spec(1, d),
                  _full_spec(1, 1), _row_spec(1)],
        out_specs=[_row_spec(d), _full_spec(g, d)],
        out_shape=[jax.ShapeDtypeStruct((n, d), jnp.float32),
                   jax.ShapeDtypeStruct((g, d), jnp.float32)],
    )(acc0, acc1, di, b, a, gid2)


def _appnp_step(acc0, acc1, di, h0, do, last):
    n, d = acc0.shape

    def body(a0, a1, di_ref, h0_ref, do_ref, o_ref):
        h = (1.0 - ALPHA) * (a0[...] + a1[...]) * di_ref[...] \
            + ALPHA * h0_ref[...]
        o_ref[...] = h if last else h * do_ref[...]

    return pl.pallas_call(
        body,
        grid=(n // _BM,),
        in_specs=[_row_spec(d), _row_spec(d), _row_spec(1), _row_spec(d),
                  _row_spec(1)],
        out_specs=_row_spec(d),
        out_shape=jax.ShapeDtypeStruct((n, d), jnp.float32),
    )(acc0, acc1, di, h0, do)


def _mlp(x, w0, b0, a0, w1, b1, a1, w2, b2, a2, ws, bs):
    n, k = x.shape
    d = w0.shape[1]
    bm = min(_BM, n)

    def body(x_ref, w0r, b0r, a0r, w1r, b1r, a1r, w2r, b2r, a2r, wsr, bsr,
             o_ref):
        xv = x_ref[...]
        h = _prelu(jnp.dot(xv, w0r[...], preferred_element_type=jnp.float32)
                   + b0r[...], a0r[0, 0])
        h = _prelu(jnp.dot(h, w1r[...], preferred_element_type=jnp.float32)
                   + b1r[...], a1r[0, 0])
        h = _prelu(jnp.dot(h, w2r[...], preferred_element_type=jnp.float32)
                   + b2r[...], a2r[0, 0])
        o_ref[...] = h + jnp.dot(xv, wsr[...],
                                 preferred_element_type=jnp.float32) + bsr[...]

    row = pl.BlockSpec((bm, k), lambda i: (i, 0))
    rowo = pl.BlockSpec((bm, d), lambda i: (i, 0))
    return pl.pallas_call(
        body,
        grid=(n // bm,),
        in_specs=[row,
                  _full_spec(k, d), _full_spec(1, d), _full_spec(1, 1),
                  _full_spec(d, d), _full_spec(1, d), _full_spec(1, 1),
                  _full_spec(d, d), _full_spec(1, d), _full_spec(1, 1),
                  _full_spec(k, d), _full_spec(1, d)],
        out_specs=rowo,
        out_shape=jax.ShapeDtypeStruct((n, d), jnp.float32),
    )(x, w0, b0, a0, w1, b1, a1, w2, b2, a2, ws, bs)


def _loss_sums(l_enc, g_enc, gid2):
    n, d = l_enc.shape
    g = g_enc.shape[0]
    log2 = 0.6931471805599453

    def body(l_ref, g_ref, gid_ref, p_ref, n_ref):
        @pl.when(pl.program_id(0) == 0)
        def _():
            p_ref[...] = jnp.zeros_like(p_ref)
            n_ref[...] = jnp.zeros_like(n_ref)

        res = lax.dot_general(l_ref[...], g_ref[...],
                              (((1,), (1,)), ((), ())),
                              preferred_element_type=jnp.float32)
        cols = lax.broadcasted_iota(jnp.int32, (_BM, g), 1)
        mask = (gid_ref[...] == cols).astype(jnp.float32)
        sp = jax.nn.softplus(-res)
        p_ref[...] += jnp.sum(mask * (log2 - sp)).reshape(1, 1)
        n_ref[...] += jnp.sum((1.0 - mask) * (sp + res - log2)).reshape(1, 1)

    return pl.pallas_call(
        body,
        grid=(n // _BM,),
        in_specs=[_row_spec(d), _full_spec(g, d), _row_spec(1)],
        out_specs=[_full_spec(1, 1), _full_spec(1, 1)],
        out_shape=[jax.ShapeDtypeStruct((1, 1), jnp.float32),
                   jax.ShapeDtypeStruct((1, 1), jnp.float32)],
    )(l_enc, g_enc, gid2)


def kernel(feat, params, edge_index, graph_id):
    n, d = feat.shape
    e = edge_index.shape[1]
    g = 128
    src = edge_index[0]
    dst = edge_index[1]
    gid2 = graph_id[:, None]

    zeros_nd = jnp.zeros((n, d), jnp.float32)
    ones_nd = jnp.ones((n, d), jnp.float32)

    prop = _make_propagate(n, d, e)
    cnt = _make_count(n, d, e)

    di0, di1 = cnt(dst, ones_nd, zeros_nd)
    do0, do1 = cnt(src, ones_nd, zeros_nd)
    do, di, feat_do = _prep(do0, do1, di0, di1, feat)

    def r2(v):
        return jnp.reshape(v, (1, 1))

    def gcn(x0, ws, bs, a, nlayers):
        segs = []
        y = _mm_scale(x0, ws[0], do)
        for l in range(nlayers):
            a0, a1 = prop(y, src, dst, zeros_nd)
            if l + 1 < nlayers:
                x, y, hg = _gcn_step(a0, a1, di, bs[l][None, :], r2(a),
                                     ws[l + 1], do, gid2, g)
            else:
                x, hg = _gcn_last(a0, a1, di, bs[l][None, :], r2(a), gid2, g)
            segs.append(hg)
        return x, segs

    p = params

    ws1, bs1, al1 = p["enc1_W"], p["enc1_b"], p["enc1_a"]
    segs1 = []
    y = _mm_scale(feat, ws1[0], do)
    hdo = feat_do
    xg = None
    for k in range(KPROP):
        pa0, pa1 = prop(hdo, src, dst, zeros_nd)
        if k < 4:
            ga0, ga1 = prop(y, src, dst, zeros_nd)
            if k < 3:
                xg, y, hgk = _gcn_step(ga0, ga1, di, bs1[k][None, :], r2(al1),
                                       ws1[k + 1], do, gid2, g)
            else:
                xg, hgk = _gcn_last(ga0, ga1, di, bs1[3][None, :], r2(al1),
                                    gid2, g)
            segs1.append(hgk)
        hdo = _appnp_step(pa0, pa1, di, feat, do, last=(k == KPROP - 1))
    local_v1 = xg
    global_v1 = jnp.concatenate(segs1, axis=-1)

    out, out_global = _mm_bias_prelu(hdo, p["ppnp_W"], p["ppnp_b"][None, :],
                                     r2(p["ppnp_a"]), gid2, g)

    local_v2, segs2 = gcn(out, p["enc2_W"], p["enc2_b"], p["enc2_a"], 3)
    global_v2 = jnp.concatenate([out_global] + segs2, axis=-1)

    lm = p["local_mlp"]
    gm = p["global_mlp"]

    def run_mlp(x, m):
        return _mlp(x, m["W0"], m["b0"][None, :], r2(m["a0"]),
                    m["W1"], m["b1"][None, :], r2(m["a1"]),
                    m["W2"], m["b2"][None, :], r2(m["a2"]),
                    m["Ws"], m["bs"][None, :])

    local_v1 = run_mlp(local_v1, lm)
    local_v2 = run_mlp(local_v2, lm)
    global_v1 = run_mlp(global_v1, gm)
    global_v2 = run_mlp(global_v2, gm)

    def lgl(l_enc, g_enc):
        ps, ns = _loss_sums(l_enc, g_enc, gid2)
        e_pos = ps[0, 0] / n
        e_neg = ns[0, 0] / (n * (g - 1))
        return e_neg - e_pos

    return lgl(local_v1, global_v2) + lgl(local_v2, global_v1)


# speedup vs baseline: 1.2202x; 1.0114x over previous
import functools

import jax
import jax.numpy as jnp
from jax import lax
from jax.experimental import pallas as pl
from jax.experimental.pallas import tpu as pltpu
from jax.experimental.pallas import tpu_sc as plsc

NC = 2
NS = 16
NW = NC * NS

ALPHA = 0.2
KPROP = 20


@functools.lru_cache(maxsize=None)
def _make_propagate(n, d, e, dtype=jnp.float32):
    epw = e // NW
    B = 128
    nfull = epw // B
    tail = epw - nfull * B
    pairs = nfull // 2
    odd = nfull - pairs * 2
    rows_a = (n // NS) // 8 * 8
    rem = n - rows_a * NS
    mesh = plsc.VectorSubcoreMesh(core_axis_name="c", subcore_axis_name="s")

    def tile_rows_copy(src, dst, s):
        pltpu.sync_copy(src.at[pl.ds(s * rows_a, rows_a)],
                        dst.at[pl.ds(s * rows_a, rows_a)])
        if rem:
            @pl.when(s == NS - 1)
            def _():
                pltpu.sync_copy(src.at[pl.ds(rows_a * NS, rem)],
                                dst.at[pl.ds(rows_a * NS, rem)])

    @functools.partial(
        pl.kernel,
        out_type=[jax.ShapeDtypeStruct((n, d), dtype) for _ in range(NC)],
        mesh=mesh,
        scratch_types=[
            pltpu.VMEM((epw,), jnp.int32),
            pltpu.VMEM((B,), jnp.int32),
            pltpu.VMEM((B,), jnp.int32),
            pltpu.VMEM((tail if tail else 8,), jnp.int32),
            pltpu.VMEM((2, B, d), dtype),
            pltpu.VMEM((tail if tail else 8, d), dtype),
            pltpu.VMEM_SHARED((n, d), dtype),
            pltpu.SemaphoreType.DMA,
            pltpu.SemaphoreType.DMA,
            pltpu.SemaphoreType.DMA,
            pltpu.SemaphoreType.DMA,
            pltpu.SemaphoreType.DMA,
        ],
    )
    def prop(h_hbm, src_hbm, dst_hbm, zeros_hbm, out0, out1,
             sidx_all, didx0, didx1, didxt, rows, rowst, acc,
             gsem0, gsem1, isem0, isem1, tsem):
        c = lax.axis_index("c")
        s = lax.axis_index("s")
        wid = s * NC + c
        tile_rows_copy(zeros_hbm, acc, s)
        base = wid * epw
        pltpu.sync_copy(src_hbm.at[pl.ds(base, epw)], sidx_all)

        didxs = (didx0, didx1)
        isems = (isem0, isem1)
        gsems = (gsem0, gsem1)

        def gather(j, p, sem):
            return pltpu.async_copy(
                h_hbm.at[sidx_all.at[pl.ds(j * B, B)]], rows.at[p], sem)

        def gather_wait(p, sem):
            pltpu.make_async_copy(
                h_hbm.at[sidx_all.at[pl.ds(0, B)]], rows.at[p], sem).wait()

        def idx_load(j, p):
            return pltpu.async_copy(
                dst_hbm.at[pl.ds(pl.multiple_of(base + j * B, 8), B)],
                didxs[p], isems[p])

        def idx_wait(p):
            pltpu.make_async_copy(dst_hbm.at[pl.ds(base, B)],
                                  didxs[p], isems[p]).wait()

        if nfull > 0:
            pltpu.sync_copy(dst_hbm.at[pl.ds(base, B)], didx0)
            gather(0, 0, gsem0)
            if nfull > 1:
                pltpu.sync_copy(dst_hbm.at[pl.ds(base + B, B)], didx1)
        if tail:
            toff = pl.multiple_of(base + nfull * B, 8)
            pltpu.sync_copy(dst_hbm.at[pl.ds(toff, tail)], didxt)
            pltpu.async_copy(
                h_hbm.at[sidx_all.at[pl.ds(nfull * B, tail)]], rowst, tsem)
        plsc.subcore_barrier()

        def half(t, j, p):
            @pl.when(j + 1 < nfull)
            def _():
                gather(j + 1, 1 - p, gsems[1 - p])

            gather_wait(p, gsems[p])

            @pl.when(t > 0)
            def _():
                idx_wait(p)

            pltpu.sync_copy(rows.at[p], acc.at[didxs[p]], add=True)

            @pl.when(j + 2 < nfull)
            def _():
                idx_load(j + 2, p)

        def pair(t, carry):
            half(t, 2 * t, 0)
            half(t, 2 * t + 1, 1)
            return carry

        lax.fori_loop(0, pairs, pair, 0)
        if odd:
            half(pairs, nfull - 1, 0)
        if tail:
            pltpu.make_async_copy(
                h_hbm.at[sidx_all.at[pl.ds(nfull * B, tail)]],
                rowst, tsem).wait()
            pltpu.sync_copy(rowst, acc.at[didxt], add=True)
        plsc.subcore_barrier()

        @pl.when(c == 0)
        def _():
            tile_rows_copy(acc, out0, s)

        @pl.when(c == 1)
        def _():
            tile_rows_copy(acc, out1, s)

    return prop


@functools.lru_cache(maxsize=None)
def _make_count(n, d, e):
    epw = e // NW
    B = 128
    nfull = epw // B
    tail = epw - nfull * B
    pairs = nfull // 2
    odd = nfull - pairs * 2
    rows_a = (n // NS) // 8 * 8
    rem = n - rows_a * NS
    mesh = plsc.VectorSubcoreMesh(core_axis_name="c", subcore_axis_name="s")

    def tile_rows_copy(src, dst, s):
        pltpu.sync_copy(src.at[pl.ds(s * rows_a, rows_a)],
                        dst.at[pl.ds(s * rows_a, rows_a)])
        if rem:
            @pl.when(s == NS - 1)
            def _():
                pltpu.sync_copy(src.at[pl.ds(rows_a * NS, rem)],
                                dst.at[pl.ds(rows_a * NS, rem)])

    @functools.partial(
        pl.kernel,
        out_type=[jax.ShapeDtypeStruct((n, d), jnp.float32)
                  for _ in range(NC)],
        mesh=mesh,
        scratch_types=[
            pltpu.VMEM((B,), jnp.int32),
            pltpu.VMEM((B,), jnp.int32),
            pltpu.VMEM((tail if tail else 8,), jnp.int32),
            pltpu.VMEM((B, d), jnp.float32),
            pltpu.VMEM_SHARED((n, d), jnp.float32),
            pltpu.SemaphoreType.DMA,
            pltpu.SemaphoreType.DMA,
        ],
    )
    def cnt(idx_hbm, ones_hbm, zeros_hbm, out0, out1,
            didx0, didx1, didxt, ones_v, acc, isem0, isem1):
        c = lax.axis_index("c")
        s = lax.axis_index("s")
        wid = s * NC + c
        tile_rows_copy(zeros_hbm, acc, s)
        pltpu.sync_copy(ones_hbm.at[pl.ds(0, B)], ones_v)
        plsc.subcore_barrier()
        base = wid * epw

        didxs = (didx0, didx1)
        isems = (isem0, isem1)

        def idx_load(j, p):
            pltpu.async_copy(
                dst_hbm_slice(j), didxs[p], isems[p])

        def dst_hbm_slice(j):
            return idx_hbm.at[pl.ds(pl.multiple_of(base + j * B, 8), B)]

        def idx_wait(p):
            pltpu.make_async_copy(idx_hbm.at[pl.ds(base, B)],
                                  didxs[p], isems[p]).wait()

        if nfull > 0:
            pltpu.sync_copy(dst_hbm_slice(0), didx0)
            if nfull > 1:
                pltpu.sync_copy(dst_hbm_slice(1), didx1)

        def half(t, j, p):
            @pl.when(t > 0)
            def _():
                idx_wait(p)

            pltpu.sync_copy(ones_v, acc.at[didxs[p]], add=True)

            @pl.when(j + 2 < nfull)
            def _():
                idx_load(j + 2, p)

        def pair(t, carry):
            half(t, 2 * t, 0)
            half(t, 2 * t + 1, 1)
            return carry

        lax.fori_loop(0, pairs, pair, 0)
        if odd:
            half(pairs, nfull - 1, 0)
        if tail:
            toff = pl.multiple_of(base + nfull * B, 8)
            pltpu.sync_copy(idx_hbm.at[pl.ds(toff, tail)], didxt)
            pltpu.sync_copy(ones_v.at[pl.ds(0, tail)], acc.at[didxt],
                            add=True)
        plsc.subcore_barrier()

        @pl.when(c == 0)
        def _():
            tile_rows_copy(acc, out0, s)

        @pl.when(c == 1)
        def _():
            tile_rows_copy(acc, out1, s)

    return cnt


_BM = 1000


def _prelu(x, a):
    return jnp.where(x >= 0, x, a * x)


def _row_spec(d):
    return pl.BlockSpec((_BM, d), lambda i: (i, 0))


def _full_spec(r, c):
    return pl.BlockSpec((r, c), lambda i: (0, 0))


def _prep(pout0, pout1, pin0, pin1, feat):
    n, d = feat.shape

    def body(po0, po1, pi0, pi1, f, do_ref, di_ref, fdo_ref):
        dego = jnp.maximum(po0[:, :1] + po1[:, :1], 1.0)
        degi = jnp.maximum(pi0[:, :1] + pi1[:, :1], 1.0)
        do = lax.rsqrt(dego)
        di = lax.rsqrt(degi)
        do_ref[...] = do
        di_ref[...] = di
        fdo_ref[...] = f[...] * do

    return pl.pallas_call(
        body,
        grid=(n // _BM,),
        in_specs=[_row_spec(d)] * 4 + [_row_spec(d)],
        out_specs=[_row_spec(1), _row_spec(1), _row_spec(d)],
        out_shape=[
            jax.ShapeDtypeStruct((n, 1), jnp.float32),
            jax.ShapeDtypeStruct((n, 1), jnp.float32),
            jax.ShapeDtypeStruct((n, d), jnp.float32),
        ],
    )(pout0, pout1, pin0, pin1, feat)


def _mm_scale(x, w, scale):
    n, k = x.shape
    d = w.shape[1]

    def body(x_ref, w_ref, s_ref, o_ref):
        o_ref[...] = jnp.dot(x_ref[...], w_ref[...],
                             preferred_element_type=jnp.float32) * s_ref[...]

    return pl.pallas_call(
        body,
        grid=(n // _BM,),
        in_specs=[_row_spec(k), _full_spec(k, d), _row_spec(1)],
        out_specs=_row_spec(d),
        out_shape=jax.ShapeDtypeStruct((n, d), jnp.float32),
    )(x, w, scale)


def _mm_bias_prelu(x, w, b, a, gid2, g):
    n, k = x.shape
    d = w.shape[1]

    def body(x_ref, w_ref, b_ref, a_ref, gid_ref, o_ref, hg_ref):
        h = jnp.dot(x_ref[...], w_ref[...],
                    preferred_element_type=jnp.float32) + b_ref[...]
        o = _prelu(h, a_ref[0, 0])
        o_ref[...] = o
        _seg_accum(hg_ref, o, gid_ref, g, pl.program_id(0))

    return pl.pallas_call(
        body,
        grid=(n // _BM,),
        in_specs=[_row_spec(k), _full_spec(k, d), _full_spec(1, d),
                  _full_spec(1, 1), _row_spec(1)],
        out_specs=[_row_spec(d), _full_spec(g, d)],
        out_shape=[jax.ShapeDtypeStruct((n, d), jnp.float32),
                   jax.ShapeDtypeStruct((g, d), jnp.float32)],
    )(x, w, b, a, gid2)


def _seg_accum(hg_ref, x, gid_ref, g, pid):
    @pl.when(pid == 0)
    def _():
        hg_ref[...] = jnp.zeros_like(hg_ref)

    cols = lax.broadcasted_iota(jnp.int32, (x.shape[0], g), 1)
    onehot = (gid_ref[...] == cols).astype(jnp.float32)
    hg_ref[...] += lax.dot_general(onehot, x, (((0,), (0,)), ((), ())),
                                   preferred_element_type=jnp.float32)


def _gcn_step(acc0, acc1, di, b, a, w, do, gid2, g):
    n, d = acc0.shape

    def body(a0, a1, di_ref, b_ref, al_ref, w_ref, do_ref, gid_ref,
             x_ref, y_ref, hg_ref):
        x = _prelu((a0[...] + a1[...]) * di_ref[...] + b_ref[...], al_ref[0, 0])
        x_ref[...] = x
        y_ref[...] = jnp.dot(x, w_ref[...],
                             preferred_element_type=jnp.float32) * do_ref[...]
        _seg_accum(hg_ref, x, gid_ref, g, pl.program_id(0))

    return pl.pallas_call(
        body,
        grid=(n // _BM,),
        in_specs=[_row_spec(d), _row_spec(d), _row_spec(1), _full_spec(1, d),
                  _full_spec(1, 1), _full_spec(d, d), _row_spec(1),
                  _row_spec(1)],
        out_specs=[_row_spec(d), _row_spec(d), _full_spec(g, d)],
        out_shape=[jax.ShapeDtypeStruct((n, d), jnp.float32),
                   jax.ShapeDtypeStruct((n, d), jnp.float32),
                   jax.ShapeDtypeStruct((g, d), jnp.float32)],
    )(acc0, acc1, di, b, a, w, do, gid2)


def _gcn_last(acc0, acc1, di, b, a, gid2, g):
    n, d = acc0.shape

    def body(a0, a1, di_ref, b_ref, al_ref, gid_ref, x_ref, hg_ref):
        x = _prelu((a0[...] + a1[...]) * di_ref[...] + b_ref[...],
                   al_ref[0, 0])
        x_ref[...] = x
        _seg_accum(hg_ref, x, gid_ref, g, pl.program_id(0))

    return pl.pallas_call(
        body,
        grid=(n // _BM,),
        in_specs=[_row_spec(d), _row_spec(d), _row_spec(1), _full_spec(1, d),
                  _full_spec(1, 1), _row_spec(1)],
        out_specs=[_row_spec(d), _full_spec(g, d)],
        out_shape=[jax.ShapeDtypeStruct((n, d), jnp.float32),
                   jax.ShapeDtypeStruct((g, d), jnp.float32)],
    )(acc0, acc1, di, b, a, gid2)


def _appnp_step(acc0, acc1, di, h0, do, last):
    n, d = acc0.shape

    def body(a0, a1, di_ref, h0_ref, do_ref, o_ref):
        h = (1.0 - ALPHA) * (a0[...] + a1[...]) * di_ref[...] \
            + ALPHA * h0_ref[...]
        o_ref[...] = h if last else h * do_ref[...]

    return pl.pallas_call(
        body,
        grid=(n // _BM,),
        in_specs=[_row_spec(d), _row_spec(d), _row_spec(1), _row_spec(d),
                  _row_spec(1)],
        out_specs=_row_spec(d),
        out_shape=jax.ShapeDtypeStruct((n, d), jnp.float32),
    )(acc0, acc1, di, h0, do)


def _mlp(x, w0, b0, a0, w1, b1, a1, w2, b2, a2, ws, bs):
    n, k = x.shape
    d = w0.shape[1]
    bm = min(_BM, n)

    def body(x_ref, w0r, b0r, a0r, w1r, b1r, a1r, w2r, b2r, a2r, wsr, bsr,
             o_ref):
        xv = x_ref[...]
        h = _prelu(jnp.dot(xv, w0r[...], preferred_element_type=jnp.float32)
                   + b0r[...], a0r[0, 0])
        h = _prelu(jnp.dot(h, w1r[...], preferred_element_type=jnp.float32)
                   + b1r[...], a1r[0, 0])
        h = _prelu(jnp.dot(h, w2r[...], preferred_element_type=jnp.float32)
                   + b2r[...], a2r[0, 0])
        o_ref[...] = h + jnp.dot(xv, wsr[...],
                                 preferred_element_type=jnp.float32) + bsr[...]

    row = pl.BlockSpec((bm, k), lambda i: (i, 0))
    rowo = pl.BlockSpec((bm, d), lambda i: (i, 0))
    return pl.pallas_call(
        body,
        grid=(n // bm,),
        in_specs=[row,
                  _full_spec(k, d), _full_spec(1, d), _full_spec(1, 1),
                  _full_spec(d, d), _full_spec(1, d), _full_spec(1, 1),
                  _full_spec(d, d), _full_spec(1, d), _full_spec(1, 1),
                  _full_spec(k, d), _full_spec(1, d)],
        out_specs=rowo,
        out_shape=jax.ShapeDtypeStruct((n, d), jnp.float32),
    )(x, w0, b0, a0, w1, b1, a1, w2, b2, a2, ws, bs)


def _loss_sums(l_enc, g_enc, gid2):
    n, d = l_enc.shape
    g = g_enc.shape[0]
    log2 = 0.6931471805599453

    def body(l_ref, g_ref, gid_ref, p_ref, n_ref):
        @pl.when(pl.program_id(0) == 0)
        def _():
            p_ref[...] = jnp.zeros_like(p_ref)
            n_ref[...] = jnp.zeros_like(n_ref)

        res = lax.dot_general(l_ref[...], g_ref[...],
                              (((1,), (1,)), ((), ())),
                              preferred_element_type=jnp.float32)
        cols = lax.broadcasted_iota(jnp.int32, (_BM, g), 1)
        mask = (gid_ref[...] == cols).astype(jnp.float32)
        sp = jax.nn.softplus(-res)
        p_ref[...] += jnp.sum(mask * (log2 - sp)).reshape(1, 1)
        n_ref[...] += jnp.sum((1.0 - mask) * (sp + res - log2)).reshape(1, 1)

    return pl.pallas_call(
        body,
        grid=(n // _BM,),
        in_specs=[_row_spec(d), _full_spec(g, d), _row_spec(1)],
        out_specs=[_full_spec(1, 1), _full_spec(1, 1)],
        out_shape=[jax.ShapeDtypeStruct((1, 1), jnp.float32),
                   jax.ShapeDtypeStruct((1, 1), jnp.float32)],
    )(l_enc, g_enc, gid2)


def kernel(feat, params, edge_index, graph_id):
    n, d = feat.shape
    e = edge_index.shape[1]
    g = 128
    src = edge_index[0]
    dst = edge_index[1]
    gid2 = graph_id[:, None]

    zeros_nd = jnp.zeros((n, d), jnp.float32)
    ones_nd = jnp.ones((n, d), jnp.float32)

    prop = _make_propagate(n, d, e)
    cnt = _make_count(n, d, e)

    di0, di1 = cnt(dst, ones_nd, zeros_nd)
    do0, do1 = cnt(src, ones_nd, zeros_nd)
    do, di, feat_do = _prep(do0, do1, di0, di1, feat)

    def r2(v):
        return jnp.reshape(v, (1, 1))

    def gcn(x0, ws, bs, a, nlayers):
        segs = []
        y = _mm_scale(x0, ws[0], do)
        for l in range(nlayers):
            a0, a1 = prop(y, src, dst, zeros_nd)
            if l + 1 < nlayers:
                x, y, hg = _gcn_step(a0, a1, di, bs[l][None, :], r2(a),
                                     ws[l + 1], do, gid2, g)
            else:
                x, hg = _gcn_last(a0, a1, di, bs[l][None, :], r2(a), gid2, g)
            segs.append(hg)
        return x, segs

    p = params

    ws1, bs1, al1 = p["enc1_W"], p["enc1_b"], p["enc1_a"]
    segs1 = []
    y = _mm_scale(feat, ws1[0], do)
    hdo = feat_do
    xg = None
    for k in range(KPROP):
        pa0, pa1 = prop(hdo, src, dst, zeros_nd)
        if k < 4:
            ga0, ga1 = prop(y, src, dst, zeros_nd)
            if k < 3:
                xg, y, hgk = _gcn_step(ga0, ga1, di, bs1[k][None, :], r2(al1),
                                       ws1[k + 1], do, gid2, g)
            else:
                xg, hgk = _gcn_last(ga0, ga1, di, bs1[3][None, :], r2(al1),
                                    gid2, g)
            segs1.append(hgk)
        hdo = _appnp_step(pa0, pa1, di, feat, do, last=(k == KPROP - 1))
    local_v1 = xg
    global_v1 = jnp.concatenate(segs1, axis=-1)

    out, out_global = _mm_bias_prelu(hdo, p["ppnp_W"], p["ppnp_b"][None, :],
                                     r2(p["ppnp_a"]), gid2, g)

    local_v2, segs2 = gcn(out, p["enc2_W"], p["enc2_b"], p["enc2_a"], 3)
    global_v2 = jnp.concatenate([out_global] + segs2, axis=-1)

    lm = p["local_mlp"]
    gm = p["global_mlp"]

    def run_mlp(x, m):
        return _mlp(x, m["W0"], m["b0"][None, :], r2(m["a0"]),
                    m["W1"], m["b1"][None, :], r2(m["a1"]),
                    m["W2"], m["b2"][None, :], r2(m["a2"]),
                    m["Ws"], m["bs"][None, :])

    local_v1 = run_mlp(local_v1, lm)
    local_v2 = run_mlp(local_v2, lm)
    global_v1 = run_mlp(global_v1, gm)
    global_v2 = run_mlp(global_v2, gm)

    def lgl(l_enc, g_enc):
        ps, ns = _loss_sums(l_enc, g_enc, gid2)
        e_pos = ps[0, 0] / n
        e_neg = ns[0, 0] / (n * (g - 1))
        return e_neg - e_pos

    return lgl(local_v1, global_v2) + lgl(local_v2, global_v1)
